# Initial kernel scaffold; baseline (speedup 1.0000x reference)
import jax
import jax.numpy as jnp
from jax.experimental import pallas as pl


def kernel(x, edge_index, W1, b1, W2, b2):
    raise NotImplementedError("write your pallas kernel here")


# speedup vs baseline: 10.2139x; 10.2139x over previous
import functools

import jax
import jax.numpy as jnp
from jax import lax
from jax.experimental import pallas as pl
from jax.experimental.pallas import tpu as pltpu
from jax.experimental.pallas import tpu_sc as plsc

NC = 2
NS = 16
L = 16

G = 64
BR = 20
EB = BR * G


def _zero16(dt):
  return jnp.zeros((L,), dt)


GH = 128
BH = EB // GH


def _make_hist(N, E):
  nblk = E // EB
  nbf = nblk // NS
  nbr = nblk % NS
  sw = ((N // NS + 7) // 8) * 8
  mesh = plsc.VectorSubcoreMesh(core_axis_name="c", subcore_axis_name="s")

  @functools.partial(
      pl.kernel,
      out_type=jax.ShapeDtypeStruct((NC * N,), jnp.float32),
      mesh=mesh,
      compiler_params=pltpu.CompilerParams(needs_layout_passes=False),
      scratch_types=[
          pltpu.VMEM((BH, GH), jnp.int32),
          pltpu.VMEM((GH,), jnp.float32),
          pltpu.VMEM((sw,), jnp.float32),
          pltpu.VMEM_SHARED((N,), jnp.float32),
      ],
  )
  def hist_kernel(dst_hbm, out_hbm, idx_v, ones_v, zbuf_v, hist_sh):
    c = lax.axis_index("c")
    t = lax.axis_index("s")

    def fill(i, _):
      ones_v[pl.ds(i * L, L)] = jnp.full((L,), 1.0, jnp.float32)
      return 0
    lax.fori_loop(0, GH // L, fill, 0)

    def zfill(i, _):
      zbuf_v[pl.ds(i * L, L)] = _zero16(jnp.float32)
      return 0
    lax.fori_loop(0, sw // L, zfill, 0)

    start = pl.multiple_of(jnp.minimum(t * sw, N - sw), 8)
    pltpu.sync_copy(zbuf_v, hist_sh.at[pl.ds(start, sw)])
    plsc.subcore_barrier()

    nbt = nbf + jnp.where(t < nbr, 1, 0) if nbr else nbf

    def body(i, _):
      pltpu.sync_copy(dst_hbm.at[i * NS + t], idx_v)
      def row(j, _):
        pltpu.sync_copy(ones_v, hist_sh.at[idx_v.at[j]], add=True)
        return 0
      lax.fori_loop(0, BH, row, 0)
      return 0
    lax.fori_loop(0, nbt, body, 0)

    plsc.subcore_barrier()
    pltpu.sync_copy(hist_sh.at[pl.ds(start, sw)], zbuf_v)
    pltpu.sync_copy(zbuf_v, out_hbm.at[pl.ds(pl.multiple_of(c * N + start, 8), sw)])

  return hist_kernel


def _make_spmm(N, E, F):
  per_core = N // NC
  cs_a = (((per_core + 1) // 2) + 7) // 8 * 8
  cs_b = per_core - cs_a
  arows = cs_a + 2 * NS
  sw = ((cs_a // NS + 7) // 8) * 8
  zr = 16
  nblk = E // EB
  nbf = nblk // NS
  nbr = nblk % NS
  selcap = EB + 2 * G
  mesh = plsc.VectorSubcoreMesh(core_axis_name="c", subcore_axis_name="s")

  @functools.partial(
      pl.kernel,
      out_type=jax.ShapeDtypeStruct((N, F), jnp.float32),
      mesh=mesh,
      compiler_params=pltpu.CompilerParams(needs_layout_passes=False),
      scratch_types=[
          pltpu.VMEM((BR, G), jnp.int32),
          pltpu.VMEM((BR, G), jnp.int32),
          pltpu.VMEM((selcap,), jnp.int32),
          pltpu.VMEM((selcap,), jnp.int32),
          pltpu.VMEM((G,), jnp.int32),
          pltpu.VMEM((G, F), jnp.float32),
          pltpu.VMEM((zr, F), jnp.float32),
          pltpu.VMEM_SHARED((arows, F), jnp.float32),
          pltpu.SemaphoreType.DMA,
      ],
  )
  def spmm_kernel(src_hbm, dst_hbm, y_hbm, out_hbm,
                  srcb_v, dstb_v, ssel_v, dsel_v, sidx_v,
                  rows_v, zbuf_v, acc_sh, sem):
    c = lax.axis_index("c")
    t = lax.axis_index("s")
    wid = t * NC + c
    nbt = nbf + jnp.where(t < nbr, 1, 0) if nbr else nbf

    def zfill(i, _):
      def zlane(j, _):
        zbuf_v[i, pl.ds(j * L, L)] = _zero16(jnp.float32)
        return 0
      lax.fori_loop(0, F // L, zlane, 0)
      return 0
    lax.fori_loop(0, zr, zfill, 0)

    lane = lax.iota(jnp.int32, L)
    pad_src = (wid * L + lane) % N

    for off, cs in ((0, cs_a), (cs_a, cs_b)):
      lo = c * per_core + off
      dump_row = jnp.full((L,), cs + 2 * t + c, jnp.int32)

      row0 = pl.multiple_of(jnp.minimum(t * sw, arows - sw), 8)
      def zcopy(i, _):
        r = pl.multiple_of(jnp.minimum(row0 + i * zr, arows - zr), 8)
        pltpu.sync_copy(zbuf_v, acc_sh.at[pl.ds(r, zr)])
        return 0
      lax.fori_loop(0, sw // zr, zcopy, 0)
      plsc.subcore_barrier()

      def fire(base):
        offb = pl.multiple_of(lax.rem(base, selcap), 8)
        def cp(q, _):
          sidx_v[pl.ds(q * L, L)] = dsel_v[pl.ds(offb + q * L, L)]
          return 0
        lax.fori_loop(0, G // L, cp, 0)
        pltpu.async_copy(y_hbm.at[ssel_v.at[pl.ds(offb, G)]],
                         rows_v, sem).wait()
        pltpu.sync_copy(rows_v, acc_sh.at[sidx_v], add=True)

      def blk_body(i, carry):
        ptr, fired = carry
        pltpu.sync_copy(src_hbm.at[i * NS + t], srcb_v)
        pltpu.sync_copy(dst_hbm.at[i * NS + t], dstb_v)
        def row_body(j, p):
          def vec_body(v, p2):
            d = dstb_v[j, pl.ds(v * L, L)]
            s_ = srcb_v[j, pl.ds(v * L, L)]
            m = (d >= lo) & (d < lo + cs)
            mi = jnp.where(m, jnp.full((L,), 1, jnp.int32),
                           jnp.full((L,), 0, jnp.int32))
            pos = lax.rem(p2 + plsc.cumsum(mi) - 1, selcap)
            plsc.store_scatter(ssel_v, [pos], s_, mask=m)
            plsc.store_scatter(dsel_v, [pos], d - lo, mask=m)
            return p2 + jnp.sum(mi)
          return lax.fori_loop(0, G // L, vec_body, p)
        ptr = lax.fori_loop(0, BR, row_body, ptr)
        nfire = (ptr - fired) // G
        def fi(q, _):
          fire(fired + q * G)
          return 0
        lax.fori_loop(0, nfire, fi, 0)
        return ptr, fired + nfire * G
      ptr, fired = lax.fori_loop(0, nbt, blk_body, (0, 0))

      for q in range(G // L):
        posp = lax.rem(ptr + q * L + lane, selcap)
        plsc.store_scatter(ssel_v, [posp], pad_src)
        plsc.store_scatter(dsel_v, [posp], dump_row)
      nfire2 = (ptr - fired + G - 1) // G
      def fi2(q, _):
        fire(fired + q * G)
        return 0
      lax.fori_loop(0, nfire2, fi2, 0)

      plsc.subcore_barrier()

      out0 = pl.multiple_of(jnp.minimum(t * sw, cs - sw), 8)
      rw = sw // 14
      def wcopy(i, _):
        o = pl.multiple_of(out0 + i * rw, 8)
        pltpu.sync_copy(acc_sh.at[pl.ds(o, rw)],
                        rows_v.at[pl.ds(0, rw)])
        pltpu.sync_copy(rows_v.at[pl.ds(0, rw)],
                        out_hbm.at[pl.ds(pl.multiple_of(lo + o, 8), rw)])
        return 0
      lax.fori_loop(0, sw // rw, wcopy, 0)
      plsc.subcore_barrier()

  return spmm_kernel


def _tc_m1(x, W1, hist, bm):
  N, D = x.shape
  H = W1.shape[1]

  def body(x_ref, w_ref, h_ref, o_ref):
    s = lax.rsqrt(1.0 + h_ref[...])
    y = jnp.dot(x_ref[...], w_ref[...], preferred_element_type=jnp.float32)
    o_ref[...] = y * s

  return pl.pallas_call(
      body,
      grid=(N // bm,),
      in_specs=[
          pl.BlockSpec((bm, D), lambda i: (i, 0)),
          pl.BlockSpec((D, H), lambda i: (0, 0)),
          pl.BlockSpec((bm, 1), lambda i: (i, 0)),
      ],
      out_specs=pl.BlockSpec((bm, H), lambda i: (i, 0)),
      out_shape=jax.ShapeDtypeStruct((N, H), jnp.float32),
  )(x, W1, hist)


def _tc_m2(acc1, y1s, hist, W2, b1, bm):
  N, H = acc1.shape
  O = W2.shape[1]

  def body(a_ref, y_ref, h_ref, w_ref, b_ref, o_ref):
    s = lax.rsqrt(1.0 + h_ref[...])
    hdn = jnp.maximum((a_ref[...] + y_ref[...]) * s + b_ref[...], 0.0)
    y2 = jnp.dot(hdn, w_ref[...], preferred_element_type=jnp.float32) * s
    o_ref[...] = jnp.concatenate(
        [y2, jnp.zeros((y2.shape[0], o_ref.shape[1] - y2.shape[1]),
                       jnp.float32)], axis=1)

  return pl.pallas_call(
      body,
      grid=(N // bm,),
      in_specs=[
          pl.BlockSpec((bm, H), lambda i: (i, 0)),
          pl.BlockSpec((bm, H), lambda i: (i, 0)),
          pl.BlockSpec((bm, 1), lambda i: (i, 0)),
          pl.BlockSpec((H, O), lambda i: (0, 0)),
          pl.BlockSpec((1, H), lambda i: (0, 0)),
      ],
      out_specs=pl.BlockSpec((bm, H), lambda i: (i, 0)),
      out_shape=jax.ShapeDtypeStruct((N, H), jnp.float32),
  )(acc1, y1s, hist, W2, b1)


def _tc_m3(acc2, y2s, hist, b2, bm):
  N, H = acc2.shape
  O = b2.shape[1]

  def body(a_ref, y_ref, h_ref, b_ref, o_ref):
    s = lax.rsqrt(1.0 + h_ref[...])
    o_ref[...] = (a_ref[:, :O] + y_ref[:, :O]) * s + b_ref[...]

  return pl.pallas_call(
      body,
      grid=(N // bm,),
      in_specs=[
          pl.BlockSpec((bm, H), lambda i: (i, 0)),
          pl.BlockSpec((bm, H), lambda i: (i, 0)),
          pl.BlockSpec((bm, 1), lambda i: (i, 0)),
          pl.BlockSpec((1, O), lambda i: (0, 0)),
      ],
      out_specs=pl.BlockSpec((bm, O), lambda i: (i, 0)),
      out_shape=jax.ShapeDtypeStruct((N, O), jnp.float32),
  )(acc2, y2s, hist, b2)


def kernel(x, edge_index, W1, b1, W2, b2):
  N, D = x.shape
  E = edge_index.shape[1]
  H = W1.shape[1]
  O = W2.shape[1]
  bm = 400

  src = edge_index[0]
  dst = edge_index[1]
  dst_blocks = dst.reshape(E // EB, BR, G)
  src_blocks = src.reshape(E // EB, BR, G)

  hist2 = _make_hist(N, E)(dst.reshape(E // EB, BH, GH))
  hist = hist2[:N][:, None]

  y1s = _tc_m1(x, W1, hist, bm)
  acc1 = _make_spmm(N, E, H)(src_blocks, dst_blocks, y1s)
  y2s = _tc_m2(acc1, y1s, hist, W2, b1.reshape(1, H), bm)
  acc2 = _make_spmm(N, E, H)(src_blocks, dst_blocks, y2s)
  return _tc_m3(acc2, y2s, hist, b2.reshape(1, O), bm)
